# baseline (speedup 1.0000x reference)
import functools

import jax
import jax.numpy as jnp
from jax import lax
from jax.experimental import pallas as pl
from jax.experimental.pallas import tpu as pltpu
from jax.experimental.pallas import tpu_sc as plsc

_TOKENS = 16384
_D = 2048
_E = 64
_S = 12288
_T = 1024
_NBUF = 4

_NC = 2
_NS = 16
_TPW = _S // (_NC * _NS)
_L = 16


def _ring(x_hbm, xbuf, sems, i, tok0, nchunk):
    def chunk_copy(j, slot):
        return pltpu.make_async_copy(
            x_hbm.at[pl.ds(tok0 + j * _T, _T), :], xbuf.at[slot],
            sems.at[slot])

    @pl.when(i == 0)
    def _prime():
        for s in range(_NBUF - 1):
            chunk_copy(s, s).start()

    pref = i + _NBUF - 1

    @pl.when(pref < nchunk)
    def _prefetch():
        chunk_copy(pref, lax.rem(pref, _NBUF)).start()

    slot = lax.rem(i, _NBUF)
    chunk_copy(i, slot).wait()
    return slot


def _logits_body(x_hbm, w_ref, b_ref, lgt_ref, xbuf, sems, *, tok0, nchunk):
    i = pl.program_id(0)
    slot = _ring(x_hbm, xbuf, sems, i, tok0, nchunk)
    logits = jax.lax.dot_general(
        xbuf[slot], w_ref[...], (((1,), (1,)), ((), ())),
        preferred_element_type=jnp.float32) + b_ref[...]
    lgt_ref[...] = logits.T


def _tc_logits_t(x, W, b, tok0, ntok):
    nchunk = ntok // _T
    return pl.pallas_call(
        functools.partial(_logits_body, tok0=tok0, nchunk=nchunk),
        grid=(nchunk,),
        in_specs=[
            pl.BlockSpec(memory_space=pl.ANY),
            pl.BlockSpec((_E, _D), lambda i: (0, 0)),
            pl.BlockSpec((1, _E), lambda i: (0, 0)),
        ],
        out_specs=pl.BlockSpec((_E, _T), lambda i: (0, i)),
        out_shape=jax.ShapeDtypeStruct((_E, ntok), jnp.float32),
        scratch_shapes=[
            pltpu.VMEM((_NBUF, _T, _D), jnp.float32),
            pltpu.SemaphoreType.DMA((_NBUF,)),
        ],
    )(x, W, b.reshape(1, _E))


def _fused_body(x_hbm, w_ref, b_ref, tw_ref, ti_ref, xbuf, sems, *, tok0,
                nchunk):
    i = pl.program_id(0)
    slot = _ring(x_hbm, xbuf, sems, i, tok0, nchunk)
    logits = jax.lax.dot_general(
        xbuf[slot], w_ref[...], (((1,), (1,)), ((), ())),
        preferred_element_type=jnp.float32) + b_ref[...]
    m = jnp.max(logits, axis=-1, keepdims=True)
    e = jnp.exp(logits - m)
    p = e / jnp.sum(e, axis=-1, keepdims=True)

    col = jax.lax.broadcasted_iota(jnp.int32, p.shape, 1)
    m1 = jnp.max(p, axis=-1, keepdims=True)
    i1 = jnp.min(jnp.where(p == m1, col, _E), axis=-1, keepdims=True)
    masked = jnp.where(col == i1, -1.0, p)
    m2 = jnp.max(masked, axis=-1, keepdims=True)
    i2 = jnp.min(jnp.where(masked == m2, col, _E), axis=-1, keepdims=True)

    tw_ref[...] = jnp.concatenate([m1, m2], axis=-1)
    ti_ref[...] = jnp.concatenate([i1, i2], axis=-1)


def _tc_fused(x, W, b, tok0, ntok):
    nchunk = ntok // _T
    return pl.pallas_call(
        functools.partial(_fused_body, tok0=tok0, nchunk=nchunk),
        grid=(nchunk,),
        in_specs=[
            pl.BlockSpec(memory_space=pl.ANY),
            pl.BlockSpec((_E, _D), lambda i: (0, 0)),
            pl.BlockSpec((1, _E), lambda i: (0, 0)),
        ],
        out_specs=[
            pl.BlockSpec((_T, 2), lambda i: (i, 0)),
            pl.BlockSpec((_T, 2), lambda i: (i, 0)),
        ],
        out_shape=[
            jax.ShapeDtypeStruct((ntok, 2), jnp.float32),
            jax.ShapeDtypeStruct((ntok, 2), jnp.int32),
        ],
        scratch_shapes=[
            pltpu.VMEM((_NBUF, _T, _D), jnp.float32),
            pltpu.SemaphoreType.DMA((_NBUF,)),
        ],
    )(x, W, b.reshape(1, _E))


def _sc_top2_body(lgt_hbm, tw_hbm, ti_hbm, buf, tws, tis, sem0, sem1):
    wid = lax.axis_index("s") * _NC + lax.axis_index("c")
    base = wid * _TPW
    half = (_TPW // 2 + 127) // 128 * 128
    rest = _TPW - half
    cp0 = pltpu.make_async_copy(
        lgt_hbm.at[:, pl.ds(base, half)], buf.at[:, pl.ds(0, half)], sem0)
    cp1 = pltpu.make_async_copy(
        lgt_hbm.at[:, pl.ds(base + half, rest)],
        buf.at[:, pl.ds(half, rest)], sem1)
    cp0.start()
    cp1.start()

    def group(g, carry):
        t0 = g * _L
        mstack = []
        for e in range(_E):
            node, lvl = buf[e, pl.ds(t0, _L)], 0
            while mstack and mstack[-1][0] == lvl:
                node = jnp.maximum(mstack.pop()[1], node)
                lvl += 1
            mstack.append((lvl, node))
        m = mstack[0][1]
        s = jnp.zeros((_L,), jnp.float32)
        v1 = jnp.full((_L,), -1.0, jnp.float32)
        i1 = jnp.zeros((_L,), jnp.int32)
        v2 = jnp.full((_L,), -1.0, jnp.float32)
        i2 = jnp.zeros((_L,), jnp.int32)
        for e in range(_E):
            ecol = jnp.full((_L,), e, jnp.int32)
            ev = jnp.exp(buf[e, pl.ds(t0, _L)] - m)
            s = s + ev
            gt1 = ev > v1
            gt2 = ev > v2
            v2 = jnp.maximum(v2, jnp.minimum(v1, ev))
            i2 = jnp.where(gt1, i1, jnp.where(gt2, ecol, i2))
            v1 = jnp.maximum(v1, ev)
            i1 = jnp.where(gt1, ecol, i1)
        tws[0, pl.ds(t0, _L)] = v1 / s
        tws[1, pl.ds(t0, _L)] = v2 / s
        tis[0, pl.ds(t0, _L)] = i1
        tis[1, pl.ds(t0, _L)] = i2
        return carry

    cp0.wait()
    lax.fori_loop(0, half // _L, group, 0)
    cp1.wait()
    lax.fori_loop(half // _L, _TPW // _L, group, 0)

    pltpu.sync_copy(tws, tw_hbm.at[:, pl.ds(base, _TPW)])
    pltpu.sync_copy(tis, ti_hbm.at[:, pl.ds(base, _TPW)])


_sc_top2 = functools.partial(
    pl.kernel,
    out_type=[
        jax.ShapeDtypeStruct((2, _S), jnp.float32),
        jax.ShapeDtypeStruct((2, _S), jnp.int32),
    ],
    mesh=plsc.VectorSubcoreMesh(
        core_axis_name="c", subcore_axis_name="s", num_cores=_NC,
        num_subcores=_NS),
    scratch_types=[
        pltpu.VMEM((_E, _TPW), jnp.float32),
        pltpu.VMEM((2, _TPW), jnp.float32),
        pltpu.VMEM((2, _TPW), jnp.int32),
        pltpu.SemaphoreType.DMA,
        pltpu.SemaphoreType.DMA,
    ],
)(_sc_top2_body)


@jax.jit
def _router(x, W, b):
    logits_t = _tc_logits_t(x, W, b, 0, _S)
    tw_sc, ti_sc = _sc_top2(logits_t)
    tw_tc, ti_tc = _tc_fused(x, W, b, _S, _TOKENS - _S)
    tw = jnp.concatenate([tw_sc.T, tw_tc], axis=0)
    ti = jnp.concatenate([ti_sc.T, ti_tc], axis=0)
    return tw, ti


def kernel(x, k, W, b):
    tw, ti = _router(x, W, b)
    ti = ti + (jnp.asarray(k, dtype=ti.dtype) - 2)
    return (tw, ti)


# speedup vs baseline: 1.0069x; 1.0069x over previous
import functools

import jax
import jax.numpy as jnp
from jax import lax
from jax.experimental import pallas as pl
from jax.experimental.pallas import tpu as pltpu
from jax.experimental.pallas import tpu_sc as plsc

_TOKENS = 16384
_D = 2048
_E = 64
_S = 16384
_T = 1024
_NBUF = 4

_NC = 2
_NS = 16
_TPW = _S // (_NC * _NS)
_L = 16


def _ring(x_hbm, xbuf, sems, i, tok0, nchunk):
    def chunk_copy(j, slot):
        return pltpu.make_async_copy(
            x_hbm.at[pl.ds(tok0 + j * _T, _T), :], xbuf.at[slot],
            sems.at[slot])

    @pl.when(i == 0)
    def _prime():
        for s in range(_NBUF - 1):
            chunk_copy(s, s).start()

    pref = i + _NBUF - 1

    @pl.when(pref < nchunk)
    def _prefetch():
        chunk_copy(pref, lax.rem(pref, _NBUF)).start()

    slot = lax.rem(i, _NBUF)
    chunk_copy(i, slot).wait()
    return slot


def _logits_body(x_hbm, w_ref, b_ref, lgt_ref, xbuf, sems, *, tok0, nchunk):
    i = pl.program_id(0)
    slot = _ring(x_hbm, xbuf, sems, i, tok0, nchunk)
    logits = jax.lax.dot_general(
        xbuf[slot], w_ref[...], (((1,), (1,)), ((), ())),
        preferred_element_type=jnp.float32) + b_ref[...]
    lgt_ref[...] = logits.T


def _tc_logits_t(x, W, b, tok0, ntok):
    nchunk = ntok // _T
    return pl.pallas_call(
        functools.partial(_logits_body, tok0=tok0, nchunk=nchunk),
        grid=(nchunk,),
        in_specs=[
            pl.BlockSpec(memory_space=pl.ANY),
            pl.BlockSpec((_E, _D), lambda i: (0, 0)),
            pl.BlockSpec((1, _E), lambda i: (0, 0)),
        ],
        out_specs=pl.BlockSpec((_E, _T), lambda i: (0, i)),
        out_shape=jax.ShapeDtypeStruct((_E, ntok), jnp.float32),
        scratch_shapes=[
            pltpu.VMEM((_NBUF, _T, _D), jnp.float32),
            pltpu.SemaphoreType.DMA((_NBUF,)),
        ],
    )(x, W, b.reshape(1, _E))


def _fused_body(x_hbm, w_ref, b_ref, tw_ref, ti_ref, xbuf, sems, *, tok0,
                nchunk):
    i = pl.program_id(0)
    slot = _ring(x_hbm, xbuf, sems, i, tok0, nchunk)
    logits = jax.lax.dot_general(
        xbuf[slot], w_ref[...], (((1,), (1,)), ((), ())),
        preferred_element_type=jnp.float32) + b_ref[...]
    m = jnp.max(logits, axis=-1, keepdims=True)
    e = jnp.exp(logits - m)
    p = e / jnp.sum(e, axis=-1, keepdims=True)

    col = jax.lax.broadcasted_iota(jnp.int32, p.shape, 1)
    m1 = jnp.max(p, axis=-1, keepdims=True)
    i1 = jnp.min(jnp.where(p == m1, col, _E), axis=-1, keepdims=True)
    masked = jnp.where(col == i1, -1.0, p)
    m2 = jnp.max(masked, axis=-1, keepdims=True)
    i2 = jnp.min(jnp.where(masked == m2, col, _E), axis=-1, keepdims=True)

    tw_ref[...] = jnp.concatenate([m1, m2], axis=-1)
    ti_ref[...] = jnp.concatenate([i1, i2], axis=-1)


def _tc_fused(x, W, b, tok0, ntok):
    nchunk = ntok // _T
    return pl.pallas_call(
        functools.partial(_fused_body, tok0=tok0, nchunk=nchunk),
        grid=(nchunk,),
        in_specs=[
            pl.BlockSpec(memory_space=pl.ANY),
            pl.BlockSpec((_E, _D), lambda i: (0, 0)),
            pl.BlockSpec((1, _E), lambda i: (0, 0)),
        ],
        out_specs=[
            pl.BlockSpec((_T, 2), lambda i: (i, 0)),
            pl.BlockSpec((_T, 2), lambda i: (i, 0)),
        ],
        out_shape=[
            jax.ShapeDtypeStruct((ntok, 2), jnp.float32),
            jax.ShapeDtypeStruct((ntok, 2), jnp.int32),
        ],
        scratch_shapes=[
            pltpu.VMEM((_NBUF, _T, _D), jnp.float32),
            pltpu.SemaphoreType.DMA((_NBUF,)),
        ],
    )(x, W, b.reshape(1, _E))


def _sc_top2_body(lgt_hbm, tw_hbm, ti_hbm, buf, tws, tis, sem0, sem1):
    wid = lax.axis_index("s") * _NC + lax.axis_index("c")
    base = wid * _TPW
    half = (_TPW // 2 + 127) // 128 * 128
    rest = _TPW - half
    cp0 = pltpu.make_async_copy(
        lgt_hbm.at[:, pl.ds(base, half)], buf.at[:, pl.ds(0, half)], sem0)
    cp1 = pltpu.make_async_copy(
        lgt_hbm.at[:, pl.ds(base + half, rest)],
        buf.at[:, pl.ds(half, rest)], sem1)
    cp0.start()
    cp1.start()

    def group(g, carry):
        t0 = g * _L
        mstack = []
        for e in range(_E):
            node, lvl = buf[e, pl.ds(t0, _L)], 0
            while mstack and mstack[-1][0] == lvl:
                node = jnp.maximum(mstack.pop()[1], node)
                lvl += 1
            mstack.append((lvl, node))
        m = mstack[0][1]
        s = jnp.zeros((_L,), jnp.float32)
        v1 = jnp.full((_L,), -1.0, jnp.float32)
        i1 = jnp.zeros((_L,), jnp.int32)
        v2 = jnp.full((_L,), -1.0, jnp.float32)
        i2 = jnp.zeros((_L,), jnp.int32)
        for e in range(_E):
            ecol = jnp.full((_L,), e, jnp.int32)
            ev = jnp.exp(buf[e, pl.ds(t0, _L)] - m)
            s = s + ev
            gt1 = ev > v1
            gt2 = ev > v2
            v2 = jnp.maximum(v2, jnp.minimum(v1, ev))
            i2 = jnp.where(gt1, i1, jnp.where(gt2, ecol, i2))
            v1 = jnp.maximum(v1, ev)
            i1 = jnp.where(gt1, ecol, i1)
        tws[0, pl.ds(t0, _L)] = v1 / s
        tws[1, pl.ds(t0, _L)] = v2 / s
        tis[0, pl.ds(t0, _L)] = i1
        tis[1, pl.ds(t0, _L)] = i2
        return carry

    cp0.wait()
    lax.fori_loop(0, half // _L, group, 0)
    cp1.wait()
    lax.fori_loop(half // _L, _TPW // _L, group, 0)

    pltpu.sync_copy(tws, tw_hbm.at[:, pl.ds(base, _TPW)])
    pltpu.sync_copy(tis, ti_hbm.at[:, pl.ds(base, _TPW)])


_sc_top2 = functools.partial(
    pl.kernel,
    out_type=[
        jax.ShapeDtypeStruct((2, _S), jnp.float32),
        jax.ShapeDtypeStruct((2, _S), jnp.int32),
    ],
    mesh=plsc.VectorSubcoreMesh(
        core_axis_name="c", subcore_axis_name="s", num_cores=_NC,
        num_subcores=_NS),
    scratch_types=[
        pltpu.VMEM((_E, _TPW), jnp.float32),
        pltpu.VMEM((2, _TPW), jnp.float32),
        pltpu.VMEM((2, _TPW), jnp.int32),
        pltpu.SemaphoreType.DMA,
        pltpu.SemaphoreType.DMA,
    ],
)(_sc_top2_body)


@jax.jit
def _router(x, W, b):
    logits_t = _tc_logits_t(x, W, b, 0, _S)
    tw_sc, ti_sc = _sc_top2(logits_t)
    return tw_sc.T, ti_sc.T


def kernel(x, k, W, b):
    tw, ti = _router(x, W, b)
    ti = ti + (jnp.asarray(k, dtype=ti.dtype) - 2)
    return (tw, ti)
